# baseline (speedup 1.0000x reference)
import functools

import jax
import jax.numpy as jnp
from jax import lax
from jax.experimental import pallas as pl
from jax.experimental.pallas import tpu as pltpu
from jax.experimental.pallas import tpu_sc as plsc

NC, NS = 2, 16
NW = NC * NS
GRP = 128
SUP = 10
SUPE = GRP * SUP
PK = 4


def _sigmoid(x):
    return jax.nn.sigmoid(x)


def _mlp_ln_v(x, w0, b0, w1, b1, g, be):
    h = jnp.dot(x, w0, preferred_element_type=jnp.float32) + b0
    h = h * _sigmoid(h)
    y = jnp.dot(h, w1, preferred_element_type=jnp.float32) + b1
    m = jnp.mean(y, axis=-1, keepdims=True)
    v = jnp.mean((y - m) ** 2, axis=-1, keepdims=True)
    return (y - m) / jnp.sqrt(v + 1e-6) * g + be


def _node_body(pn_ref, rs_ref,
               pe_w0, pe_b0, pe_w1, pe_b1, pe_g, pe_be,
               up_w0, up_b0, up_w1, up_b1, up_g, up_be,
               re_w0, re_b0, re_w1, re_b1, re_g, re_be,
               w0s_ref, w0r_ref, b0u_ref,
               p_out_ref, pw_out_ref, re_out_ref, rw_out_ref):
    pn = pn_ref[...]
    p = _mlp_ln_v(pn, pe_w0[...], pe_b0[...], pe_w1[...], pe_b1[...],
                  pe_g[...], pe_be[...])
    p_out_ref[...] = p + _mlp_ln_v(p, up_w0[...], up_b0[...], up_w1[...],
                                   up_b1[...], up_g[...], up_be[...])
    pw_out_ref[...] = jnp.dot(p, w0s_ref[...], preferred_element_type=jnp.float32)
    rs = rs_ref[...]
    r = _mlp_ln_v(rs, re_w0[...], re_b0[...], re_w1[...], re_b1[...],
                  re_g[...], re_be[...])
    re_out_ref[...] = r
    rw_out_ref[...] = (jnp.dot(r, w0r_ref[...], preferred_element_type=jnp.float32)
                       + b0u_ref[...])


def _gln(y, G, GT, g4, be4):
    m = jnp.dot(y, G, preferred_element_type=jnp.float32) * (1.0 / 32.0)
    xc = y - jnp.dot(m, GT, preferred_element_type=jnp.float32)
    v = jnp.dot(xc * xc, G, preferred_element_type=jnp.float32) * (1.0 / 32.0)
    inv = 1.0 / jnp.sqrt(v + 1e-6)
    return xc * jnp.dot(inv, GT, preferred_element_type=jnp.float32) * g4 + be4


def _edge_embed_l0_body(ef32_ref, w0_ref, b0_ref, out_ref):
    h = jnp.dot(ef32_ref[...], w0_ref[...], preferred_element_type=jnp.float32) \
        + b0_ref[...]
    h = h * _sigmoid(h)
    for q in range(8):
        out_ref[q, :, :] = h[:, 128 * q:128 * (q + 1)]


def _edge_body(h_ref, sc_ref, rc_ref,
               ee_w1, ee_b1, ee_g, ee_be,
               w0e_ref, ue_w1, ue_b1, ue_g, ue_be,
               G_ref, GT_ref,
               out_ref):
    G, GT = G_ref[...], GT_ref[...]
    y = jnp.dot(h_ref[...], ee_w1[...], preferred_element_type=jnp.float32) \
        + ee_b1[...]
    e = _gln(y, G, GT, ee_g[...], ee_be[...])
    x = (jnp.dot(e, w0e_ref[...], preferred_element_type=jnp.float32)
         + sc_ref[...] + rc_ref[...])
    h2 = x * _sigmoid(x)
    y2 = jnp.dot(h2, ue_w1[...], preferred_element_type=jnp.float32) + ue_b1[...]
    out_ref[...] = e + _gln(y2, G, GT, ue_g[...], ue_be[...])


def _rout_body(parts_ref, cnts_ref, re_ref,
               ur_w0, ur_b0, ur_w1, ur_b1, ur_g, ur_be,
               r_out_ref):
    ps = parts_ref[...]
    cs = cnts_ref[...]
    cnt = jnp.maximum(cs[0][:, 0:1] + cs[1][:, 0:1], 1.0)
    agg = (ps[0] + ps[1]) / cnt
    r = re_ref[...]
    x = jnp.concatenate([r, agg], axis=-1)
    r_out_ref[...] = r + _mlp_ln_v(x, ur_w0[...], ur_b0[...], ur_w1[...],
                                   ur_b1[...], ur_g[...], ur_be[...])


def _make_gather(E, L):
    nsup = E // SUPE
    iters = (nsup + NW - 1) // NW
    mesh = plsc.VectorSubcoreMesh(core_axis_name="c", subcore_axis_name="s",
                                  num_cores=NC, num_subcores=NS)

    @functools.partial(
        pl.kernel,
        out_type=[jax.ShapeDtypeStruct((E, L), jnp.float32),
                  jax.ShapeDtypeStruct((E, L), jnp.float32)],
        mesh=mesh,
        scratch_types=[
            pltpu.VMEM((SUP, GRP), jnp.int32),
            pltpu.VMEM((SUP, GRP), jnp.int32),
            pltpu.VMEM((SUPE, L), jnp.float32),
            pltpu.VMEM((SUPE, L), jnp.float32),
            pltpu.SemaphoreType.DMA,
            pltpu.SemaphoreType.DMA,
        ],
        compiler_params=pltpu.CompilerParams(use_tc_tiling_on_sc=False),
    )
    def gather(pw_hbm, rw_hbm, s2d_hbm, r2d_hbm, sent_hbm, recv_hbm,
               sidx_v, ridx_v, a_v, b_v, sem_a, sem_b):
        w = lax.axis_index("s") * NC + lax.axis_index("c")

        def body(i, carry):
            t = w + i * NW

            @pl.when(t < nsup)
            def _():
                pltpu.sync_copy(s2d_hbm.at[pl.ds(t * SUP, SUP)], sidx_v)
                pltpu.sync_copy(r2d_hbm.at[pl.ds(t * SUP, SUP)], ridx_v)
                for j in range(SUP):
                    pltpu.async_copy(pw_hbm.at[sidx_v.at[j]],
                                     a_v.at[pl.ds(j * GRP, GRP)], sem_a)
                    pltpu.async_copy(rw_hbm.at[ridx_v.at[j]],
                                     b_v.at[pl.ds(j * GRP, GRP)], sem_b)
                for j in range(SUP):
                    pltpu.make_async_copy(pw_hbm.at[sidx_v.at[j]],
                                          a_v.at[pl.ds(j * GRP, GRP)], sem_a).wait()
                    pltpu.make_async_copy(rw_hbm.at[ridx_v.at[j]],
                                          b_v.at[pl.ds(j * GRP, GRP)], sem_b).wait()
                pltpu.sync_copy(a_v, sent_hbm.at[pl.ds(t * SUPE, SUPE)])
                pltpu.sync_copy(b_v, recv_hbm.at[pl.ds(t * SUPE, SUPE)])
            return carry

        lax.fori_loop(0, iters, body, 0)

    return gather


def _make_scatter(E, NR, L, CW):
    nsup = E // SUPE
    iters = (nsup + NW - 1) // NW
    mesh = plsc.VectorSubcoreMesh(core_axis_name="c", subcore_axis_name="s",
                                  num_cores=NC, num_subcores=NS)

    @functools.partial(
        pl.kernel,
        out_type=[jax.ShapeDtypeStruct((NC, NR, L), jnp.float32),
                  jax.ShapeDtypeStruct((NC, NR, CW), jnp.float32)],
        mesh=mesh,
        scratch_types=[
            pltpu.VMEM((SUP, GRP), jnp.int32),
            pltpu.VMEM((SUPE, L), jnp.float32),
            pltpu.VMEM((GRP, CW), jnp.float32),
            pltpu.VMEM((NR, L), jnp.float32),
            pltpu.VMEM((NR, CW), jnp.float32),
            pltpu.VMEM_SHARED((NR, L), jnp.float32),
            pltpu.VMEM_SHARED((NR, CW), jnp.float32),
            pltpu.SemaphoreType.DMA,
        ],
        compiler_params=pltpu.CompilerParams(use_tc_tiling_on_sc=False),
    )
    def scatter(e_hbm, r2d_hbm, out_hbm, cnt_hbm,
                ridx_v, ev, ones_v, zv, zcv, bins, cbins, sem):
        c = lax.axis_index("c")
        s = lax.axis_index("s")
        w = s * NC + c

        one = jnp.ones((16,), jnp.float32)

        def onesrow(i, carry):
            ones_v[i, pl.ds(0, CW)] = one[0:CW] if CW < 16 else one
            return carry

        lax.fori_loop(0, GRP, onesrow, 0)

        @pl.when(s == 0)
        def _():
            zero = jnp.zeros((16,), jnp.float32)

            def zrow(i, carry):
                for k in range(L // 16):
                    zv[i, pl.ds(k * 16, 16)] = zero
                zcv[i, pl.ds(0, CW)] = zero[0:CW] if CW < 16 else zero
                return carry

            lax.fori_loop(0, NR, zrow, 0)
            pltpu.sync_copy(zv, bins)
            pltpu.sync_copy(zcv, cbins)

        plsc.subcore_barrier()

        def body(i, carry):
            t = w + i * NW

            @pl.when(t < nsup)
            def _():
                pltpu.sync_copy(r2d_hbm.at[pl.ds(t * SUP, SUP)], ridx_v)
                pltpu.sync_copy(e_hbm.at[pl.ds(t * SUPE, SUPE)], ev)
                for j in range(SUP):
                    pltpu.async_copy(ev.at[pl.ds(j * GRP, GRP)],
                                     bins.at[ridx_v.at[j]], sem, add=True)
                    pltpu.async_copy(ones_v,
                                     cbins.at[ridx_v.at[j]], sem, add=True)
                for j in range(SUP):
                    pltpu.make_async_copy(ev.at[pl.ds(j * GRP, GRP)],
                                          bins.at[ridx_v.at[j]], sem).wait()
                    pltpu.make_async_copy(ones_v,
                                          cbins.at[ridx_v.at[j]], sem).wait()
            return carry

        lax.fori_loop(0, iters, body, 0)
        plsc.subcore_barrier()

        @pl.when(s == 0)
        def _():
            pltpu.sync_copy(bins, out_hbm.at[c])
            pltpu.sync_copy(cbins, cnt_hbm.at[c])

    return scatter


def _r2(a):
    return a.reshape(1, -1)


def _bd(w):
    return jnp.kron(jnp.eye(PK, dtype=w.dtype), w)


def _t4(b):
    return jnp.tile(b, PK).reshape(1, -1)


def kernel(pnode_features, pnode_struct, rnode_struct, edge_features, params,
           senders, receivers, tau):
    NP_, B, FIN = pnode_features.shape
    NR = rnode_struct.shape[0]
    E = edge_features.shape[0]
    L = params["embed_edge_w1"].shape[1]
    f32 = jnp.float32

    pn = jnp.concatenate([pnode_features[:, 0, :].astype(f32),
                          pnode_struct.astype(f32)], axis=-1)
    rs = rnode_struct.astype(f32)
    FE = edge_features.shape[1]
    ef32 = edge_features.astype(f32).reshape(E // 32, 32 * FE)
    sq = senders.astype(jnp.int32).reshape(E // 32, 8, PK).swapaxes(0, 1).reshape(E)
    rq = receivers.astype(jnp.int32).reshape(E // 32, 8, PK).swapaxes(0, 1).reshape(E)
    s2d = sq.reshape(E // GRP, GRP)
    r2d = rq.reshape(E // GRP, GRP)
    p = params
    ue_w0 = p["upd_edge_w0"]

    def w6(name):
        return (p[name + "_w0"], _r2(p[name + "_b0"]), p[name + "_w1"],
                _r2(p[name + "_b1"]), _r2(p[name + "_g"]), _r2(p[name + "_be"]))

    def w6p(name):
        return (_bd(p[name + "_w0"]), _t4(p[name + "_b0"]), _bd(p[name + "_w1"]),
                _t4(p[name + "_b1"]), _t4(p[name + "_g"]), _t4(p[name + "_be"]))

    node_in = (pn, rs) + w6("embed_pnode") + w6("upd_pnode") \
        + (p["embed_rnode_w0"][FIN:],) + w6("embed_rnode")[1:] \
        + (ue_w0[L:2 * L], ue_w0[2 * L:3 * L], _r2(p["upd_edge_b0"]))
    p_final, pw_t, r_embed, rw_t = pl.pallas_call(
        _node_body,
        out_shape=[jax.ShapeDtypeStruct((NP_, L), f32),
                   jax.ShapeDtypeStruct((NP_, L), f32),
                   jax.ShapeDtypeStruct((NR, L), f32),
                   jax.ShapeDtypeStruct((NR, L), f32)],
    )(*node_in)

    sent_c, recv_c = _make_gather(E, L)(pw_t, rw_t, s2d, r2d)
    sent4 = sent_c.reshape(E // PK, PK * L)
    recv4 = recv_c.reshape(E // PK, PK * L)

    E32 = E // 32
    B32 = 1000
    w0bd32 = jnp.kron(jnp.eye(32, dtype=f32), p["embed_edge_w0"])
    b0t32 = jnp.tile(p["embed_edge_b0"], 32).reshape(1, -1)
    h8 = pl.pallas_call(
        _edge_embed_l0_body,
        grid=(E32 // B32,),
        in_specs=[pl.BlockSpec((B32, 32 * FE), lambda i: (i, 0)),
                  pl.BlockSpec(w0bd32.shape, lambda i: (0, 0)),
                  pl.BlockSpec(b0t32.shape, lambda i: (0, 0))],
        out_specs=pl.BlockSpec((8, B32, PK * L), lambda i: (0, i, 0)),
        out_shape=jax.ShapeDtypeStruct((8, E32, PK * L), f32),
    )(ef32, w0bd32, b0t32)
    h4 = h8.reshape(E // PK, PK * L)

    EP = E // PK
    EB = 2000
    G = jnp.kron(jnp.eye(PK, dtype=f32), jnp.ones((L, 1), f32))
    GT = G.T
    edge_in = (h4, sent4, recv4) + w6p("embed_edge")[2:] \
        + (_bd(ue_w0[:L]),) + w6p("upd_edge")[2:] + (G, GT)
    wspecs = [pl.BlockSpec(a.shape, lambda i: (0, 0)) for a in edge_in[3:]]
    e4 = pl.pallas_call(
        _edge_body,
        grid=(EP // EB,),
        in_specs=[pl.BlockSpec((EB, PK * L), lambda i: (i, 0)),
                  pl.BlockSpec((EB, PK * L), lambda i: (i, 0)),
                  pl.BlockSpec((EB, PK * L), lambda i: (i, 0))] + wspecs,
        out_specs=pl.BlockSpec((EB, PK * L), lambda i: (i, 0)),
        out_shape=jax.ShapeDtypeStruct((EP, PK * L), f32),
    )(*edge_in)

    CW = 16
    parts, cnts = _make_scatter(E, NR, L, CW)(e4.reshape(E, L), r2d)

    rout_in = (parts, cnts, r_embed) + w6("upd_rnode")
    r_final = pl.pallas_call(
        _rout_body,
        out_shape=jax.ShapeDtypeStruct((NR, L), f32),
    )(*rout_in)

    dt = pnode_features.dtype
    return (r_final.astype(dt)[:, None, :], p_final.astype(dt)[:, None, :])


# speedup vs baseline: 1.1708x; 1.1708x over previous
import functools

import jax
import jax.numpy as jnp
from jax import lax
from jax.experimental import pallas as pl
from jax.experimental.pallas import tpu as pltpu
from jax.experimental.pallas import tpu_sc as plsc

NC, NS = 2, 16
NW = NC * NS
GRP = 128
SUP = 10
SUPE = GRP * SUP
PK = 4


def _sigmoid(x):
    return jax.nn.sigmoid(x)


def _mlp_ln_v(x, w0, b0, w1, b1, g, be):
    h = jnp.dot(x, w0, preferred_element_type=jnp.float32) + b0
    h = h * _sigmoid(h)
    y = jnp.dot(h, w1, preferred_element_type=jnp.float32) + b1
    m = jnp.mean(y, axis=-1, keepdims=True)
    v = jnp.mean((y - m) ** 2, axis=-1, keepdims=True)
    return (y - m) / jnp.sqrt(v + 1e-6) * g + be


def _node_body(pn_ref, rs_ref,
               pe_w0, pe_b0, pe_w1, pe_b1, pe_g, pe_be,
               up_w0, up_b0, up_w1, up_b1, up_g, up_be,
               re_w0, re_b0, re_w1, re_b1, re_g, re_be,
               w0s_ref, w0r_ref, b0u_ref,
               p_out_ref, pw_out_ref, re_out_ref, rw_out_ref):
    pn = pn_ref[...]
    p = _mlp_ln_v(pn, pe_w0[...], pe_b0[...], pe_w1[...], pe_b1[...],
                  pe_g[...], pe_be[...])
    p_out_ref[...] = p + _mlp_ln_v(p, up_w0[...], up_b0[...], up_w1[...],
                                   up_b1[...], up_g[...], up_be[...])
    pw_out_ref[...] = jnp.dot(
        p, w0s_ref[...], preferred_element_type=jnp.float32).astype(jnp.bfloat16)
    rs = rs_ref[...]
    r = _mlp_ln_v(rs, re_w0[...], re_b0[...], re_w1[...], re_b1[...],
                  re_g[...], re_be[...])
    re_out_ref[...] = r
    rw_out_ref[...] = (jnp.dot(r, w0r_ref[...], preferred_element_type=jnp.float32)
                       + b0u_ref[...]).astype(jnp.bfloat16)


def _gln(y, G, GT, g4, be4):
    m = jnp.dot(y, G, preferred_element_type=jnp.float32) * (1.0 / 32.0)
    xc = y - jnp.dot(m, GT, preferred_element_type=jnp.float32)
    v = jnp.dot(xc * xc, G, preferred_element_type=jnp.float32) * (1.0 / 32.0)
    inv = 1.0 / jnp.sqrt(v + 1e-6)
    return xc * jnp.dot(inv, GT, preferred_element_type=jnp.float32) * g4 + be4


def _edge_body(ef_ref, sc_ref, rc_ref,
               ee_w0, ee_b0, ee_w1, ee_b1, ee_g, ee_be,
               w0e_ref, ue_w1, ue_b1, ue_g, ue_be,
               G_ref, GT_ref,
               out_ref):
    G, GT = G_ref[...], GT_ref[...]
    h = jnp.dot(ef_ref[...], ee_w0[...], preferred_element_type=jnp.float32) \
        + ee_b0[...]
    h = h * _sigmoid(h)
    y = jnp.dot(h, ee_w1[...], preferred_element_type=jnp.float32) + ee_b1[...]
    e = _gln(y, G, GT, ee_g[...], ee_be[...])
    x = (jnp.dot(e, w0e_ref[...], preferred_element_type=jnp.float32)
         + sc_ref[...].astype(jnp.float32) + rc_ref[...].astype(jnp.float32))
    h2 = x * _sigmoid(x)
    y2 = jnp.dot(h2, ue_w1[...], preferred_element_type=jnp.float32) + ue_b1[...]
    out_ref[...] = e + _gln(y2, G, GT, ue_g[...], ue_be[...])


def _rout_body(parts_ref, cnts_ref, re_ref,
               ur_w0, ur_b0, ur_w1, ur_b1, ur_g, ur_be,
               r_out_ref):
    ps = parts_ref[...]
    cs = cnts_ref[...]
    cnt = jnp.maximum(cs[0][:, 0:1] + cs[1][:, 0:1], 1.0)
    agg = (ps[0] + ps[1]) / cnt
    r = re_ref[...]
    x = jnp.concatenate([r, agg], axis=-1)
    r_out_ref[...] = r + _mlp_ln_v(x, ur_w0[...], ur_b0[...], ur_w1[...],
                                   ur_b1[...], ur_g[...], ur_be[...])


def _make_gather(E, L):
    nsup = E // SUPE
    iters = (nsup + NW - 1) // NW
    mesh = plsc.VectorSubcoreMesh(core_axis_name="c", subcore_axis_name="s",
                                  num_cores=NC, num_subcores=NS)

    @functools.partial(
        pl.kernel,
        out_type=[jax.ShapeDtypeStruct((E, L), jnp.bfloat16),
                  jax.ShapeDtypeStruct((E, L), jnp.bfloat16)],
        mesh=mesh,
        scratch_types=[
            pltpu.VMEM((SUP, GRP), jnp.int32),
            pltpu.VMEM((SUP, GRP), jnp.int32),
            pltpu.VMEM((SUPE, L), jnp.bfloat16),
            pltpu.VMEM((SUPE, L), jnp.bfloat16),
            pltpu.SemaphoreType.DMA,
            pltpu.SemaphoreType.DMA,
        ],
        compiler_params=pltpu.CompilerParams(use_tc_tiling_on_sc=False),
    )
    def gather(pw_hbm, rw_hbm, s2d_hbm, r2d_hbm, sent_hbm, recv_hbm,
               sidx_v, ridx_v, a_v, b_v, sem_a, sem_b):
        w = lax.axis_index("s") * NC + lax.axis_index("c")

        def body(i, carry):
            t = w + i * NW

            @pl.when(t < nsup)
            def _():
                pltpu.sync_copy(s2d_hbm.at[pl.ds(t * SUP, SUP)], sidx_v)
                pltpu.sync_copy(r2d_hbm.at[pl.ds(t * SUP, SUP)], ridx_v)
                for j in range(SUP):
                    pltpu.async_copy(pw_hbm.at[sidx_v.at[j]],
                                     a_v.at[pl.ds(j * GRP, GRP)], sem_a)
                    pltpu.async_copy(rw_hbm.at[ridx_v.at[j]],
                                     b_v.at[pl.ds(j * GRP, GRP)], sem_b)
                for j in range(SUP):
                    pltpu.make_async_copy(pw_hbm.at[sidx_v.at[j]],
                                          a_v.at[pl.ds(j * GRP, GRP)], sem_a).wait()
                    pltpu.make_async_copy(rw_hbm.at[ridx_v.at[j]],
                                          b_v.at[pl.ds(j * GRP, GRP)], sem_b).wait()
                pltpu.sync_copy(a_v, sent_hbm.at[pl.ds(t * SUPE, SUPE)])
                pltpu.sync_copy(b_v, recv_hbm.at[pl.ds(t * SUPE, SUPE)])
            return carry

        lax.fori_loop(0, iters, body, 0)

    return gather


def _make_scatter(E, NR, L, CW):
    nsup = E // SUPE
    iters = (nsup + NW - 1) // NW
    mesh = plsc.VectorSubcoreMesh(core_axis_name="c", subcore_axis_name="s",
                                  num_cores=NC, num_subcores=NS)

    @functools.partial(
        pl.kernel,
        out_type=[jax.ShapeDtypeStruct((NC, NR, L), jnp.float32),
                  jax.ShapeDtypeStruct((NC, NR, CW), jnp.float32)],
        mesh=mesh,
        scratch_types=[
            pltpu.VMEM((SUP, GRP), jnp.int32),
            pltpu.VMEM((SUPE, L), jnp.float32),
            pltpu.VMEM((GRP, CW), jnp.float32),
            pltpu.VMEM((NR, L), jnp.float32),
            pltpu.VMEM((NR, CW), jnp.float32),
            pltpu.VMEM_SHARED((NR, L), jnp.float32),
            pltpu.VMEM_SHARED((NR, CW), jnp.float32),
            pltpu.SemaphoreType.DMA,
        ],
        compiler_params=pltpu.CompilerParams(use_tc_tiling_on_sc=False),
    )
    def scatter(e_hbm, r2d_hbm, out_hbm, cnt_hbm,
                ridx_v, ev, ones_v, zv, zcv, bins, cbins, sem):
        c = lax.axis_index("c")
        s = lax.axis_index("s")
        w = s * NC + c

        one = jnp.ones((16,), jnp.float32)

        def onesrow(i, carry):
            ones_v[i, pl.ds(0, CW)] = one[0:CW] if CW < 16 else one
            return carry

        lax.fori_loop(0, GRP, onesrow, 0)

        @pl.when(s == 0)
        def _():
            zero = jnp.zeros((16,), jnp.float32)

            def zrow(i, carry):
                for k in range(L // 16):
                    zv[i, pl.ds(k * 16, 16)] = zero
                zcv[i, pl.ds(0, CW)] = zero[0:CW] if CW < 16 else zero
                return carry

            lax.fori_loop(0, NR, zrow, 0)
            pltpu.sync_copy(zv, bins)
            pltpu.sync_copy(zcv, cbins)

        plsc.subcore_barrier()

        def body(i, carry):
            t = w + i * NW

            @pl.when(t < nsup)
            def _():
                pltpu.sync_copy(r2d_hbm.at[pl.ds(t * SUP, SUP)], ridx_v)
                pltpu.sync_copy(e_hbm.at[pl.ds(t * SUPE, SUPE)], ev)
                for j in range(SUP):
                    pltpu.async_copy(ev.at[pl.ds(j * GRP, GRP)],
                                     bins.at[ridx_v.at[j]], sem, add=True)
                    pltpu.async_copy(ones_v,
                                     cbins.at[ridx_v.at[j]], sem, add=True)
                for j in range(SUP):
                    pltpu.make_async_copy(ev.at[pl.ds(j * GRP, GRP)],
                                          bins.at[ridx_v.at[j]], sem).wait()
                    pltpu.make_async_copy(ones_v,
                                          cbins.at[ridx_v.at[j]], sem).wait()
            return carry

        lax.fori_loop(0, iters, body, 0)
        plsc.subcore_barrier()

        @pl.when(s == 0)
        def _():
            pltpu.sync_copy(bins, out_hbm.at[c])
            pltpu.sync_copy(cbins, cnt_hbm.at[c])

    return scatter


def _r2(a):
    return a.reshape(1, -1)


def _bd(w):
    return jnp.kron(jnp.eye(PK, dtype=w.dtype), w)


def _t4(b):
    return jnp.tile(b, PK).reshape(1, -1)


def kernel(pnode_features, pnode_struct, rnode_struct, edge_features, params,
           senders, receivers, tau):
    NP_, B, FIN = pnode_features.shape
    NR = rnode_struct.shape[0]
    E = edge_features.shape[0]
    L = params["embed_edge_w1"].shape[1]
    f32 = jnp.float32

    pn = jnp.concatenate([pnode_features[:, 0, :].astype(f32),
                          pnode_struct.astype(f32)], axis=-1)
    rs = rnode_struct.astype(f32)
    FE = edge_features.shape[1]
    ef4 = jnp.concatenate([edge_features.astype(f32)[k::PK] for k in range(PK)],
                          axis=1)
    s2d = senders.astype(jnp.int32).reshape(E // GRP, GRP)
    r2d = receivers.astype(jnp.int32).reshape(E // GRP, GRP)
    p = params
    ue_w0 = p["upd_edge_w0"]

    def w6(name):
        return (p[name + "_w0"], _r2(p[name + "_b0"]), p[name + "_w1"],
                _r2(p[name + "_b1"]), _r2(p[name + "_g"]), _r2(p[name + "_be"]))

    def w6p(name):
        return (_bd(p[name + "_w0"]), _t4(p[name + "_b0"]), _bd(p[name + "_w1"]),
                _t4(p[name + "_b1"]), _t4(p[name + "_g"]), _t4(p[name + "_be"]))

    node_in = (pn, rs) + w6("embed_pnode") + w6("upd_pnode") \
        + (p["embed_rnode_w0"][FIN:],) + w6("embed_rnode")[1:] \
        + (ue_w0[L:2 * L], ue_w0[2 * L:3 * L], _r2(p["upd_edge_b0"]))
    p_final, pw_t, r_embed, rw_t = pl.pallas_call(
        _node_body,
        out_shape=[jax.ShapeDtypeStruct((NP_, L), f32),
                   jax.ShapeDtypeStruct((NP_, L), jnp.bfloat16),
                   jax.ShapeDtypeStruct((NR, L), f32),
                   jax.ShapeDtypeStruct((NR, L), jnp.bfloat16)],
    )(*node_in)

    sent_c, recv_c = _make_gather(E, L)(pw_t, rw_t, s2d, r2d)
    sent4 = sent_c.reshape(E // PK, PK * L)
    recv4 = recv_c.reshape(E // PK, PK * L)

    EP = E // PK
    EB = 2000
    G = jnp.kron(jnp.eye(PK, dtype=f32), jnp.ones((L, 1), f32))
    GT = G.T
    edge_in = (ef4, sent4, recv4) + w6p("embed_edge") \
        + (_bd(ue_w0[:L]),) + w6p("upd_edge")[2:] + (G, GT)
    wspecs = [pl.BlockSpec(a.shape, lambda i: (0, 0)) for a in edge_in[3:]]
    e4 = pl.pallas_call(
        _edge_body,
        grid=(EP // EB,),
        in_specs=[pl.BlockSpec((EB, ef4.shape[1]), lambda i: (i, 0)),
                  pl.BlockSpec((EB, PK * L), lambda i: (i, 0)),
                  pl.BlockSpec((EB, PK * L), lambda i: (i, 0))] + wspecs,
        out_specs=pl.BlockSpec((EB, PK * L), lambda i: (i, 0)),
        out_shape=jax.ShapeDtypeStruct((EP, PK * L), f32),
    )(*edge_in)

    CW = 16
    parts, cnts = _make_scatter(E, NR, L, CW)(e4.reshape(E, L), r2d)

    rout_in = (parts, cnts, r_embed) + w6("upd_rnode")
    r_final = pl.pallas_call(
        _rout_body,
        out_shape=jax.ShapeDtypeStruct((NR, L), f32),
    )(*rout_in)

    dt = pnode_features.dtype
    return (r_final.astype(dt)[:, None, :], p_final.astype(dt)[:, None, :])


# speedup vs baseline: 1.6494x; 1.4089x over previous
import functools

import jax
import jax.numpy as jnp
from jax import lax
from jax.experimental import pallas as pl
from jax.experimental.pallas import tpu as pltpu
from jax.experimental.pallas import tpu_sc as plsc

NC, NS = 2, 16
NW = NC * NS
GRP = 128
SUP = 10
SUPE = GRP * SUP
PK = 4


def _sigmoid(x):
    return jax.nn.sigmoid(x)


def _mlp_ln_v(x, w0, b0, w1, b1, g, be):
    h = jnp.dot(x, w0, preferred_element_type=jnp.float32) + b0
    h = h * _sigmoid(h)
    y = jnp.dot(h, w1, preferred_element_type=jnp.float32) + b1
    m = jnp.mean(y, axis=-1, keepdims=True)
    v = jnp.mean((y - m) ** 2, axis=-1, keepdims=True)
    return (y - m) / jnp.sqrt(v + 1e-6) * g + be


def _node_body(pn_ref, rs_ref,
               pe_w0, pe_b0, pe_w1, pe_b1, pe_g, pe_be,
               up_w0, up_b0, up_w1, up_b1, up_g, up_be,
               re_w0, re_b0, re_w1, re_b1, re_g, re_be,
               w0s_ref, w0r_ref, b0u_ref,
               p_out_ref, pw_out_ref, re_out_ref, rw_out_ref):
    pn = pn_ref[...]
    p = _mlp_ln_v(pn, pe_w0[...], pe_b0[...], pe_w1[...], pe_b1[...],
                  pe_g[...], pe_be[...])
    p_out_ref[...] = p + _mlp_ln_v(p, up_w0[...], up_b0[...], up_w1[...],
                                   up_b1[...], up_g[...], up_be[...])
    pw_out_ref[...] = jnp.dot(p, w0s_ref[...], preferred_element_type=jnp.float32)
    rs = rs_ref[...]
    r = _mlp_ln_v(rs, re_w0[...], re_b0[...], re_w1[...], re_b1[...],
                  re_g[...], re_be[...])
    re_out_ref[...] = r
    rw_out_ref[...] = (jnp.dot(r, w0r_ref[...], preferred_element_type=jnp.float32)
                       + b0u_ref[...])


def _gln(y, G, GT, g4, be4):
    m = jnp.dot(y, G, preferred_element_type=jnp.float32) * (1.0 / 32.0)
    xc = y - jnp.dot(m, GT, preferred_element_type=jnp.float32)
    v = jnp.dot(xc * xc, G, preferred_element_type=jnp.float32) * (1.0 / 32.0)
    inv = 1.0 / jnp.sqrt(v + 1e-6)
    return xc * jnp.dot(inv, GT, preferred_element_type=jnp.float32) * g4 + be4


def _edge_body(ef_ref, sc_ref, rc_ref,
               ee_w0, ee_b0, ee_w1, ee_b1, ee_g, ee_be,
               w0e_ref, ue_w1, ue_b1, ue_g, ue_be,
               G_ref, GT_ref,
               out_ref):
    G, GT = G_ref[...], GT_ref[...]
    h = jnp.dot(ef_ref[...], ee_w0[...], preferred_element_type=jnp.float32) \
        + ee_b0[...]
    h = h * _sigmoid(h)
    y = jnp.dot(h, ee_w1[...], preferred_element_type=jnp.float32) + ee_b1[...]
    e = _gln(y, G, GT, ee_g[...], ee_be[...])
    x = (jnp.dot(e, w0e_ref[...], preferred_element_type=jnp.float32)
         + sc_ref[...].astype(jnp.float32) + rc_ref[...].astype(jnp.float32))
    h2 = x * _sigmoid(x)
    y2 = jnp.dot(h2, ue_w1[...], preferred_element_type=jnp.float32) + ue_b1[...]
    out_ref[...] = e + _gln(y2, G, GT, ue_g[...], ue_be[...])


def _rout_body(parts_ref, cnts_ref, re_ref,
               ur_w0, ur_b0, ur_w1, ur_b1, ur_g, ur_be,
               r_out_ref):
    ps = parts_ref[...]
    cs = cnts_ref[...]
    cnt = jnp.maximum(cs[0][:, 0:1] + cs[1][:, 0:1], 1.0)
    agg = (ps[0] + ps[1]) / cnt
    r = re_ref[...]
    x = jnp.concatenate([r, agg], axis=-1)
    r_out_ref[...] = r + _mlp_ln_v(x, ur_w0[...], ur_b0[...], ur_w1[...],
                                   ur_b1[...], ur_g[...], ur_be[...])


def _make_gather(E, L):
    nsup = E // SUPE
    iters = (nsup + NW - 1) // NW
    mesh = plsc.VectorSubcoreMesh(core_axis_name="c", subcore_axis_name="s",
                                  num_cores=NC, num_subcores=NS)

    @functools.partial(
        pl.kernel,
        out_type=[jax.ShapeDtypeStruct((E, L), jnp.float32),
                  jax.ShapeDtypeStruct((E, L), jnp.float32)],
        mesh=mesh,
        scratch_types=[
            pltpu.VMEM((SUP, GRP), jnp.int32),
            pltpu.VMEM((SUP, GRP), jnp.int32),
            pltpu.VMEM((SUPE, L), jnp.float32),
            pltpu.VMEM((SUPE, L), jnp.float32),
            pltpu.SemaphoreType.DMA,
            pltpu.SemaphoreType.DMA,
        ],
        compiler_params=pltpu.CompilerParams(use_tc_tiling_on_sc=False),
    )
    def gather(pw_hbm, rw_hbm, s2d_hbm, r2d_hbm, sent_hbm, recv_hbm,
               sidx_v, ridx_v, a_v, b_v, sem_a, sem_b):
        w = lax.axis_index("s") * NC + lax.axis_index("c")

        def body(i, carry):
            t = w + i * NW

            @pl.when(t < nsup)
            def _():
                pltpu.sync_copy(s2d_hbm.at[pl.ds(t * SUP, SUP)], sidx_v)
                pltpu.sync_copy(r2d_hbm.at[pl.ds(t * SUP, SUP)], ridx_v)
                for j in range(SUP):
                    pltpu.async_copy(pw_hbm.at[sidx_v.at[j]],
                                     a_v.at[pl.ds(j * GRP, GRP)], sem_a)
                    pltpu.async_copy(rw_hbm.at[ridx_v.at[j]],
                                     b_v.at[pl.ds(j * GRP, GRP)], sem_b)
                for j in range(SUP):
                    pltpu.make_async_copy(pw_hbm.at[sidx_v.at[j]],
                                          a_v.at[pl.ds(j * GRP, GRP)], sem_a).wait()
                    pltpu.make_async_copy(rw_hbm.at[ridx_v.at[j]],
                                          b_v.at[pl.ds(j * GRP, GRP)], sem_b).wait()
                pltpu.sync_copy(a_v, sent_hbm.at[pl.ds(t * SUPE, SUPE)])
                pltpu.sync_copy(b_v, recv_hbm.at[pl.ds(t * SUPE, SUPE)])
            return carry

        lax.fori_loop(0, iters, body, 0)

    return gather


def _make_scatter(E, NR, L, CW):
    nsup = E // SUPE
    iters = (nsup + NW - 1) // NW
    mesh = plsc.VectorSubcoreMesh(core_axis_name="c", subcore_axis_name="s",
                                  num_cores=NC, num_subcores=NS)

    @functools.partial(
        pl.kernel,
        out_type=[jax.ShapeDtypeStruct((NC, NR, L), jnp.float32),
                  jax.ShapeDtypeStruct((NC, NR, CW), jnp.float32)],
        mesh=mesh,
        scratch_types=[
            pltpu.VMEM((SUP, GRP), jnp.int32),
            pltpu.VMEM((SUPE, L), jnp.float32),
            pltpu.VMEM((GRP, CW), jnp.float32),
            pltpu.VMEM((NR, L), jnp.float32),
            pltpu.VMEM((NR, CW), jnp.float32),
            pltpu.VMEM_SHARED((NR, L), jnp.float32),
            pltpu.VMEM_SHARED((NR, CW), jnp.float32),
            pltpu.SemaphoreType.DMA,
        ],
        compiler_params=pltpu.CompilerParams(use_tc_tiling_on_sc=False),
    )
    def scatter(e_hbm, r2d_hbm, out_hbm, cnt_hbm,
                ridx_v, ev, ones_v, zv, zcv, bins, cbins, sem):
        c = lax.axis_index("c")
        s = lax.axis_index("s")
        w = s * NC + c

        one = jnp.ones((16,), jnp.float32)

        def onesrow(i, carry):
            ones_v[i, pl.ds(0, CW)] = one[0:CW] if CW < 16 else one
            return carry

        lax.fori_loop(0, GRP, onesrow, 0)

        @pl.when(s == 0)
        def _():
            zero = jnp.zeros((16,), jnp.float32)

            def zrow(i, carry):
                for k in range(L // 16):
                    zv[i, pl.ds(k * 16, 16)] = zero
                zcv[i, pl.ds(0, CW)] = zero[0:CW] if CW < 16 else zero
                return carry

            lax.fori_loop(0, NR, zrow, 0)
            pltpu.sync_copy(zv, bins)
            pltpu.sync_copy(zcv, cbins)

        plsc.subcore_barrier()

        def body(i, carry):
            t = w + i * NW

            @pl.when(t < nsup)
            def _():
                pltpu.sync_copy(r2d_hbm.at[pl.ds(t * SUP, SUP)], ridx_v)
                pltpu.sync_copy(e_hbm.at[pl.ds(t * SUPE, SUPE)], ev)
                for j in range(SUP):
                    pltpu.async_copy(ev.at[pl.ds(j * GRP, GRP)],
                                     bins.at[ridx_v.at[j]], sem, add=True)
                    pltpu.async_copy(ones_v,
                                     cbins.at[ridx_v.at[j]], sem, add=True)
                for j in range(SUP):
                    pltpu.make_async_copy(ev.at[pl.ds(j * GRP, GRP)],
                                          bins.at[ridx_v.at[j]], sem).wait()
                    pltpu.make_async_copy(ones_v,
                                          cbins.at[ridx_v.at[j]], sem).wait()
            return carry

        lax.fori_loop(0, iters, body, 0)
        plsc.subcore_barrier()

        @pl.when(s == 0)
        def _():
            pltpu.sync_copy(bins, out_hbm.at[c])
            pltpu.sync_copy(cbins, cnt_hbm.at[c])

    return scatter


def _r2(a):
    return a.reshape(1, -1)


def _bd(w):
    return jnp.kron(jnp.eye(PK, dtype=w.dtype), w)


def _t4(b):
    return jnp.tile(b, PK).reshape(1, -1)


def kernel(pnode_features, pnode_struct, rnode_struct, edge_features, params,
           senders, receivers, tau):
    NP_, B, FIN = pnode_features.shape
    NR = rnode_struct.shape[0]
    E = edge_features.shape[0]
    L = params["embed_edge_w1"].shape[1]
    f32 = jnp.float32

    pn = jnp.concatenate([pnode_features[:, 0, :].astype(f32),
                          pnode_struct.astype(f32)], axis=-1)
    rs = rnode_struct.astype(f32)
    FE = edge_features.shape[1]
    ef4 = jnp.concatenate([edge_features.astype(f32)[k::PK] for k in range(PK)],
                          axis=1)
    s2d = senders.astype(jnp.int32).reshape(E // GRP, GRP)
    r2d = receivers.astype(jnp.int32).reshape(E // GRP, GRP)
    p = params
    ue_w0 = p["upd_edge_w0"]

    def w6(name):
        return (p[name + "_w0"], _r2(p[name + "_b0"]), p[name + "_w1"],
                _r2(p[name + "_b1"]), _r2(p[name + "_g"]), _r2(p[name + "_be"]))

    def w6p(name):
        return (_bd(p[name + "_w0"]), _t4(p[name + "_b0"]), _bd(p[name + "_w1"]),
                _t4(p[name + "_b1"]), _t4(p[name + "_g"]), _t4(p[name + "_be"]))

    node_in = (pn, rs) + w6("embed_pnode") + w6("upd_pnode") \
        + (p["embed_rnode_w0"][FIN:],) + w6("embed_rnode")[1:] \
        + (ue_w0[L:2 * L], ue_w0[2 * L:3 * L], _r2(p["upd_edge_b0"]))
    p_final, pw_t, r_embed, rw_t = pl.pallas_call(
        _node_body,
        out_shape=[jax.ShapeDtypeStruct((NP_, L), f32),
                   jax.ShapeDtypeStruct((NP_, L), f32),
                   jax.ShapeDtypeStruct((NR, L), f32),
                   jax.ShapeDtypeStruct((NR, L), f32)],
    )(*node_in)

    sent_c, recv_c = _make_gather(E, L)(pw_t, rw_t, s2d, r2d)
    sent4 = sent_c.reshape(E // PK, PK * L)
    recv4 = recv_c.reshape(E // PK, PK * L)

    EP = E // PK
    EB = 2000
    G = jnp.kron(jnp.eye(PK, dtype=f32), jnp.ones((L, 1), f32))
    GT = G.T
    edge_in = (ef4, sent4, recv4) + w6p("embed_edge") \
        + (_bd(ue_w0[:L]),) + w6p("upd_edge")[2:] + (G, GT)
    wspecs = [pl.BlockSpec(a.shape, lambda i: (0, 0)) for a in edge_in[3:]]
    e4 = pl.pallas_call(
        _edge_body,
        grid=(EP // EB,),
        in_specs=[pl.BlockSpec((EB, ef4.shape[1]), lambda i: (i, 0)),
                  pl.BlockSpec((EB, PK * L), lambda i: (i, 0)),
                  pl.BlockSpec((EB, PK * L), lambda i: (i, 0))] + wspecs,
        out_specs=pl.BlockSpec((EB, PK * L), lambda i: (i, 0)),
        out_shape=jax.ShapeDtypeStruct((EP, PK * L), f32),
    )(*edge_in)

    CW = 16
    parts, cnts = _make_scatter(E, NR, L, CW)(e4.reshape(E, L), r2d)

    rout_in = (parts, cnts, r_embed) + w6("upd_rnode")
    r_final = pl.pallas_call(
        _rout_body,
        out_shape=jax.ShapeDtypeStruct((NR, L), f32),
    )(*rout_in)

    dt = pnode_features.dtype
    return (r_final.astype(dt)[:, None, :], p_final.astype(dt)[:, None, :])


# speedup vs baseline: 1.6553x; 1.0035x over previous
import functools

import jax
import jax.numpy as jnp
from jax import lax
from jax.experimental import pallas as pl
from jax.experimental.pallas import tpu as pltpu
from jax.experimental.pallas import tpu_sc as plsc

NC, NS = 2, 16
NW = NC * NS
GRP = 128
SUP = 10
SUPE = GRP * SUP
PK = 4


def _sigmoid(x):
    return jax.nn.sigmoid(x)


def _mlp_ln_v(x, w0, b0, w1, b1, g, be):
    h = jnp.dot(x, w0, preferred_element_type=jnp.float32) + b0
    h = h * _sigmoid(h)
    y = jnp.dot(h, w1, preferred_element_type=jnp.float32) + b1
    m = jnp.mean(y, axis=-1, keepdims=True)
    v = jnp.mean((y - m) ** 2, axis=-1, keepdims=True)
    return (y - m) / jnp.sqrt(v + 1e-6) * g + be


def _node_body(pn_ref, rs_ref,
               pe_w0, pe_b0, pe_w1, pe_b1, pe_g, pe_be,
               up_w0, up_b0, up_w1, up_b1, up_g, up_be,
               re_w0, re_b0, re_w1, re_b1, re_g, re_be,
               w0s_ref, w0r_ref, b0u_ref,
               p_out_ref, pw_out_ref, re_out_ref, rw_out_ref):
    pn = pn_ref[...]
    p = _mlp_ln_v(pn, pe_w0[...], pe_b0[...], pe_w1[...], pe_b1[...],
                  pe_g[...], pe_be[...])
    p_out_ref[...] = p + _mlp_ln_v(p, up_w0[...], up_b0[...], up_w1[...],
                                   up_b1[...], up_g[...], up_be[...])
    pw_out_ref[...] = jnp.dot(p, w0s_ref[...], preferred_element_type=jnp.float32)
    rs = rs_ref[...]
    r = _mlp_ln_v(rs, re_w0[...], re_b0[...], re_w1[...], re_b1[...],
                  re_g[...], re_be[...])
    re_out_ref[...] = r
    rw_out_ref[...] = (jnp.dot(r, w0r_ref[...], preferred_element_type=jnp.float32)
                       + b0u_ref[...])


def _gln(y, G, GT, g4, be4):
    m = jnp.dot(y, G, preferred_element_type=jnp.float32) * (1.0 / 32.0)
    xc = y - jnp.dot(m, GT, preferred_element_type=jnp.float32)
    v = jnp.dot(xc * xc, G, preferred_element_type=jnp.float32) * (1.0 / 32.0)
    inv = 1.0 / jnp.sqrt(v + 1e-6)
    return xc * jnp.dot(inv, GT, preferred_element_type=jnp.float32) * g4 + be4


def _edge_body(ef_ref, sc_ref,
               ee_w0, ee_b0, ee_w1, ee_b1, ee_g, ee_be,
               w0e_ref, ue_w1, ue_b1, ue_g, ue_be,
               G_ref, GT_ref,
               out_ref):
    G, GT = G_ref[...], GT_ref[...]
    h = jnp.dot(ef_ref[...], ee_w0[...], preferred_element_type=jnp.float32) \
        + ee_b0[...]
    h = h * _sigmoid(h)
    y = jnp.dot(h, ee_w1[...], preferred_element_type=jnp.float32) + ee_b1[...]
    e = _gln(y, G, GT, ee_g[...], ee_be[...])
    x = (jnp.dot(e, w0e_ref[...], preferred_element_type=jnp.float32)
         + sc_ref[...])
    h2 = x * _sigmoid(x)
    y2 = jnp.dot(h2, ue_w1[...], preferred_element_type=jnp.float32) + ue_b1[...]
    out_ref[...] = e + _gln(y2, G, GT, ue_g[...], ue_be[...])


def _rout_body(parts_ref, cnts_ref, re_ref,
               ur_w0, ur_b0, ur_w1, ur_b1, ur_g, ur_be,
               r_out_ref):
    ps = parts_ref[...]
    cs = cnts_ref[...]
    cnt = jnp.maximum(cs[0][:, 0:1] + cs[1][:, 0:1], 1.0)
    agg = (ps[0] + ps[1]) / cnt
    r = re_ref[...]
    x = jnp.concatenate([r, agg], axis=-1)
    r_out_ref[...] = r + _mlp_ln_v(x, ur_w0[...], ur_b0[...], ur_w1[...],
                                   ur_b1[...], ur_g[...], ur_be[...])


def _make_gather(E, L):
    nsup = E // SUPE
    iters = (nsup + NW - 1) // NW
    mesh = plsc.VectorSubcoreMesh(core_axis_name="c", subcore_axis_name="s",
                                  num_cores=NC, num_subcores=NS)

    @functools.partial(
        pl.kernel,
        out_type=jax.ShapeDtypeStruct((E, L), jnp.float32),
        mesh=mesh,
        scratch_types=[
            pltpu.VMEM((SUP, GRP), jnp.int32),
            pltpu.VMEM((SUP, GRP), jnp.int32),
            pltpu.VMEM((SUPE, L), jnp.float32),
            pltpu.VMEM((SUPE, L), jnp.float32),
            pltpu.SemaphoreType.DMA,
            pltpu.SemaphoreType.DMA,
        ],
        compiler_params=pltpu.CompilerParams(use_tc_tiling_on_sc=False),
    )
    def gather(pw_hbm, rw_hbm, s2d_hbm, r2d_hbm, sent_hbm,
               sidx_v, ridx_v, a_v, b_v, sem_a, sem_b):
        w = lax.axis_index("s") * NC + lax.axis_index("c")

        def body(i, carry):
            t = w + i * NW

            @pl.when(t < nsup)
            def _():
                pltpu.sync_copy(s2d_hbm.at[pl.ds(t * SUP, SUP)], sidx_v)
                pltpu.sync_copy(r2d_hbm.at[pl.ds(t * SUP, SUP)], ridx_v)
                for j in range(SUP):
                    pltpu.async_copy(pw_hbm.at[sidx_v.at[j]],
                                     a_v.at[pl.ds(j * GRP, GRP)], sem_a)
                    pltpu.async_copy(rw_hbm.at[ridx_v.at[j]],
                                     b_v.at[pl.ds(j * GRP, GRP)], sem_b)
                for j in range(SUP):
                    pltpu.make_async_copy(pw_hbm.at[sidx_v.at[j]],
                                          a_v.at[pl.ds(j * GRP, GRP)], sem_a).wait()
                    pltpu.make_async_copy(rw_hbm.at[ridx_v.at[j]],
                                          b_v.at[pl.ds(j * GRP, GRP)], sem_b).wait()
                def addrow(r_, carry_):
                    for half in range(2):
                        sl = pl.ds(half * 16, 16)
                        a_v[r_, sl] = a_v[r_, sl] + b_v[r_, sl]
                    return carry_

                lax.fori_loop(0, SUPE, addrow, 0)
                pltpu.sync_copy(a_v, sent_hbm.at[pl.ds(t * SUPE, SUPE)])
            return carry

        lax.fori_loop(0, iters, body, 0)

    return gather


def _make_scatter(E, NR, L, CW):
    nsup = E // SUPE
    iters = (nsup + NW - 1) // NW
    mesh = plsc.VectorSubcoreMesh(core_axis_name="c", subcore_axis_name="s",
                                  num_cores=NC, num_subcores=NS)

    @functools.partial(
        pl.kernel,
        out_type=[jax.ShapeDtypeStruct((NC, NR, L), jnp.float32),
                  jax.ShapeDtypeStruct((NC, NR, CW), jnp.float32)],
        mesh=mesh,
        scratch_types=[
            pltpu.VMEM((SUP, GRP), jnp.int32),
            pltpu.VMEM((SUPE, L), jnp.float32),
            pltpu.VMEM((GRP, CW), jnp.float32),
            pltpu.VMEM((NR, L), jnp.float32),
            pltpu.VMEM((NR, CW), jnp.float32),
            pltpu.VMEM_SHARED((NR, L), jnp.float32),
            pltpu.VMEM_SHARED((NR, CW), jnp.float32),
            pltpu.SemaphoreType.DMA,
        ],
        compiler_params=pltpu.CompilerParams(use_tc_tiling_on_sc=False),
    )
    def scatter(e_hbm, r2d_hbm, out_hbm, cnt_hbm,
                ridx_v, ev, ones_v, zv, zcv, bins, cbins, sem):
        c = lax.axis_index("c")
        s = lax.axis_index("s")
        w = s * NC + c

        one = jnp.ones((16,), jnp.float32)

        def onesrow(i, carry):
            ones_v[i, pl.ds(0, CW)] = one[0:CW] if CW < 16 else one
            return carry

        lax.fori_loop(0, GRP, onesrow, 0)

        @pl.when(s == 0)
        def _():
            zero = jnp.zeros((16,), jnp.float32)

            def zrow(i, carry):
                for k in range(L // 16):
                    zv[i, pl.ds(k * 16, 16)] = zero
                zcv[i, pl.ds(0, CW)] = zero[0:CW] if CW < 16 else zero
                return carry

            lax.fori_loop(0, NR, zrow, 0)
            pltpu.sync_copy(zv, bins)
            pltpu.sync_copy(zcv, cbins)

        plsc.subcore_barrier()

        def body(i, carry):
            t = w + i * NW

            @pl.when(t < nsup)
            def _():
                pltpu.sync_copy(r2d_hbm.at[pl.ds(t * SUP, SUP)], ridx_v)
                pltpu.sync_copy(e_hbm.at[pl.ds(t * SUPE, SUPE)], ev)
                for j in range(SUP):
                    pltpu.async_copy(ev.at[pl.ds(j * GRP, GRP)],
                                     bins.at[ridx_v.at[j]], sem, add=True)
                    pltpu.async_copy(ones_v,
                                     cbins.at[ridx_v.at[j]], sem, add=True)
                for j in range(SUP):
                    pltpu.make_async_copy(ev.at[pl.ds(j * GRP, GRP)],
                                          bins.at[ridx_v.at[j]], sem).wait()
                    pltpu.make_async_copy(ones_v,
                                          cbins.at[ridx_v.at[j]], sem).wait()
            return carry

        lax.fori_loop(0, iters, body, 0)
        plsc.subcore_barrier()

        @pl.when(s == 0)
        def _():
            pltpu.sync_copy(bins, out_hbm.at[c])
            pltpu.sync_copy(cbins, cnt_hbm.at[c])

    return scatter


def _r2(a):
    return a.reshape(1, -1)


def _bd(w):
    return jnp.kron(jnp.eye(PK, dtype=w.dtype), w)


def _t4(b):
    return jnp.tile(b, PK).reshape(1, -1)


def kernel(pnode_features, pnode_struct, rnode_struct, edge_features, params,
           senders, receivers, tau):
    NP_, B, FIN = pnode_features.shape
    NR = rnode_struct.shape[0]
    E = edge_features.shape[0]
    L = params["embed_edge_w1"].shape[1]
    f32 = jnp.float32

    pn = jnp.concatenate([pnode_features[:, 0, :].astype(f32),
                          pnode_struct.astype(f32)], axis=-1)
    rs = rnode_struct.astype(f32)
    FE = edge_features.shape[1]
    ef4 = jnp.concatenate([edge_features.astype(f32)[k::PK] for k in range(PK)],
                          axis=1)
    s2d = senders.astype(jnp.int32).reshape(E // GRP, GRP)
    r2d = receivers.astype(jnp.int32).reshape(E // GRP, GRP)
    p = params
    ue_w0 = p["upd_edge_w0"]

    def w6(name):
        return (p[name + "_w0"], _r2(p[name + "_b0"]), p[name + "_w1"],
                _r2(p[name + "_b1"]), _r2(p[name + "_g"]), _r2(p[name + "_be"]))

    def w6p(name):
        return (_bd(p[name + "_w0"]), _t4(p[name + "_b0"]), _bd(p[name + "_w1"]),
                _t4(p[name + "_b1"]), _t4(p[name + "_g"]), _t4(p[name + "_be"]))

    node_in = (pn, rs) + w6("embed_pnode") + w6("upd_pnode") \
        + (p["embed_rnode_w0"][FIN:],) + w6("embed_rnode")[1:] \
        + (ue_w0[L:2 * L], ue_w0[2 * L:3 * L], _r2(p["upd_edge_b0"]))
    p_final, pw_t, r_embed, rw_t = pl.pallas_call(
        _node_body,
        out_shape=[jax.ShapeDtypeStruct((NP_, L), f32),
                   jax.ShapeDtypeStruct((NP_, L), f32),
                   jax.ShapeDtypeStruct((NR, L), f32),
                   jax.ShapeDtypeStruct((NR, L), f32)],
    )(*node_in)

    g_c = _make_gather(E, L)(pw_t, rw_t, s2d, r2d)
    g4 = g_c.reshape(E // PK, PK * L)

    EP = E // PK
    EB = 2000
    G = jnp.kron(jnp.eye(PK, dtype=f32), jnp.ones((L, 1), f32))
    GT = G.T
    edge_in = (ef4, g4) + w6p("embed_edge") \
        + (_bd(ue_w0[:L]),) + w6p("upd_edge")[2:] + (G, GT)
    wspecs = [pl.BlockSpec(a.shape, lambda i: (0, 0)) for a in edge_in[2:]]
    e4 = pl.pallas_call(
        _edge_body,
        grid=(EP // EB,),
        in_specs=[pl.BlockSpec((EB, ef4.shape[1]), lambda i: (i, 0)),
                  pl.BlockSpec((EB, PK * L), lambda i: (i, 0))] + wspecs,
        out_specs=pl.BlockSpec((EB, PK * L), lambda i: (i, 0)),
        out_shape=jax.ShapeDtypeStruct((EP, PK * L), f32),
    )(*edge_in)

    CW = 16
    parts, cnts = _make_scatter(E, NR, L, CW)(e4.reshape(E, L), r2d)

    rout_in = (parts, cnts, r_embed) + w6("upd_rnode")
    r_final = pl.pallas_call(
        _rout_body,
        out_shape=jax.ShapeDtypeStruct((NR, L), f32),
    )(*rout_in)

    dt = pnode_features.dtype
    return (r_final.astype(dt)[:, None, :], p_final.astype(dt)[:, None, :])
